# baseline (speedup 1.0000x reference)
from functools import partial

import numpy as np
import jax
import jax.numpy as jnp
from jax.experimental import pallas as pl
from jax.experimental.pallas import tpu as pltpu

_D_MODEL = 1024
_VALUE_RANGE = 4
_ROW_TILE = 4096


def _pos_code_sum(k, d_model):
    positions = np.arange(k, dtype=np.float64)[:, None]
    i = np.arange(0, d_model, 2, dtype=np.float64)
    omega = 1.0 / (10000.0 ** (i / d_model))
    angles = positions * omega[None, :]
    codes = np.zeros((k, d_model), np.float64)
    codes[:, 0::2] = np.sin(angles)
    codes[:, 1::2] = np.cos(angles)
    return codes.sum(axis=0).astype(np.float32)


def _prep_body(a_ref, bm_ref, w_ref, o_ref, *, k):
    raw = jax.lax.dot_general(a_ref[...], w_ref[...],
                              (((1,), (1,)), ((), ())),
                              preferred_element_type=jnp.float32)
    base = raw[4:5] + bm_ref[...] + float(k) * raw[0:1]
    deltas = raw[1:4] - raw[0:1]
    zeros = jnp.zeros((4, raw.shape[1]), jnp.float32)
    o_ref[...] = jnp.concatenate([base, deltas, zeros], axis=0)


def _main_body(ch_ref, p_ref, o_ref, *, tile, slab=8):
    d = p_ref.shape[1]
    base = jnp.broadcast_to(p_ref[0:1, :], (slab, d))
    d1 = jnp.broadcast_to(p_ref[1:2, :], (slab, d))
    d2 = jnp.broadcast_to(p_ref[2:3, :], (slab, d))
    d3 = jnp.broadcast_to(p_ref[3:4, :], (slab, d))

    for i in range(tile // slab):
        r = i * slab
        ch = jnp.clip(ch_ref[pl.ds(r, slab), :], 0, _VALUE_RANGE - 1)
        c1 = jnp.sum((ch == 1).astype(jnp.float32), axis=1, keepdims=True)
        c2 = jnp.sum((ch == 2).astype(jnp.float32), axis=1, keepdims=True)
        c3 = jnp.sum((ch == 3).astype(jnp.float32), axis=1, keepdims=True)
        o_ref[pl.ds(r, slab), :] = base + c1 * d1 + c2 * d2 + c3 * d3


def kernel(channels, value_emb, read_W, read_b, mix):
    B, L, K = channels.shape
    N = B * L
    ch2d = channels.reshape(N, K)
    mixf = jnp.asarray(mix, jnp.float32)
    pos_sum = jnp.asarray(_pos_code_sum(K, _D_MODEL))
    a = jnp.concatenate(
        [value_emb, pos_sum[None, :], jnp.zeros((3, _D_MODEL), jnp.float32)],
        axis=0) * mixf
    bm = (read_b * mixf)[None, :]

    p = pl.pallas_call(
        partial(_prep_body, k=K),
        out_shape=jax.ShapeDtypeStruct((8, _D_MODEL), jnp.float32),
    )(a, bm, read_W)

    T = _ROW_TILE
    out2d = pl.pallas_call(
        partial(_main_body, tile=T),
        grid=(N // T,),
        in_specs=[pl.BlockSpec((T, K), lambda i: (i, 0)),
                  pl.BlockSpec((8, _D_MODEL), lambda i: (0, 0))],
        out_specs=pl.BlockSpec((T, _D_MODEL), lambda i: (i, 0)),
        out_shape=jax.ShapeDtypeStruct((N, _D_MODEL), jnp.float32),
        compiler_params=pltpu.CompilerParams(
            dimension_semantics=("parallel",)),
    )(ch2d, p)
    return out2d.reshape(B, L, _D_MODEL)


# speedup vs baseline: 1.0432x; 1.0432x over previous
from functools import partial

import numpy as np
import jax
import jax.numpy as jnp
from jax.experimental import pallas as pl
from jax.experimental.pallas import tpu as pltpu

_D_MODEL = 1024
_VALUE_RANGE = 4
_ROW_TILE = 1024


def _pos_code_sum(k, d_model):
    positions = np.arange(k, dtype=np.float64)[:, None]
    i = np.arange(0, d_model, 2, dtype=np.float64)
    omega = 1.0 / (10000.0 ** (i / d_model))
    angles = positions * omega[None, :]
    codes = np.zeros((k, d_model), np.float64)
    codes[:, 0::2] = np.sin(angles)
    codes[:, 1::2] = np.cos(angles)
    return codes.sum(axis=0).astype(np.float32)


def _prep_body(a_ref, bm_ref, w_ref, o_ref, *, k):
    raw = jax.lax.dot_general(a_ref[...], w_ref[...],
                              (((1,), (1,)), ((), ())),
                              preferred_element_type=jnp.float32)
    base = raw[4:5] + bm_ref[...] + float(k) * raw[0:1]
    deltas = raw[1:4] - raw[0:1]
    zeros = jnp.zeros((4, raw.shape[1]), jnp.float32)
    o_ref[...] = jnp.concatenate([base, deltas, zeros], axis=0)


def _main_body(ch_ref, p_ref, o_ref, *, tile, slab=8):
    d = p_ref.shape[1]
    base = jnp.broadcast_to(p_ref[0:1, :], (slab, d))
    d1 = jnp.broadcast_to(p_ref[1:2, :], (slab, d))
    d2 = jnp.broadcast_to(p_ref[2:3, :], (slab, d))
    d3 = jnp.broadcast_to(p_ref[3:4, :], (slab, d))

    for i in range(tile // slab):
        r = i * slab
        ch = jnp.clip(ch_ref[pl.ds(r, slab), :], 0, _VALUE_RANGE - 1)
        c1 = jnp.sum((ch == 1).astype(jnp.float32), axis=1, keepdims=True)
        c2 = jnp.sum((ch == 2).astype(jnp.float32), axis=1, keepdims=True)
        c3 = jnp.sum((ch == 3).astype(jnp.float32), axis=1, keepdims=True)
        o_ref[pl.ds(r, slab), :] = base + c1 * d1 + c2 * d2 + c3 * d3


def kernel(channels, value_emb, read_W, read_b, mix):
    B, L, K = channels.shape
    N = B * L
    ch2d = channels.reshape(N, K)
    mixf = jnp.asarray(mix, jnp.float32)
    pos_sum = jnp.asarray(_pos_code_sum(K, _D_MODEL))
    a = jnp.concatenate(
        [value_emb, pos_sum[None, :], jnp.zeros((3, _D_MODEL), jnp.float32)],
        axis=0) * mixf
    bm = (read_b * mixf)[None, :]

    p = pl.pallas_call(
        partial(_prep_body, k=K),
        out_shape=jax.ShapeDtypeStruct((8, _D_MODEL), jnp.float32),
    )(a, bm, read_W)

    T = _ROW_TILE
    out2d = pl.pallas_call(
        partial(_main_body, tile=T),
        grid=(N // T,),
        in_specs=[pl.BlockSpec((T, K), lambda i: (i, 0)),
                  pl.BlockSpec((8, _D_MODEL), lambda i: (0, 0))],
        out_specs=pl.BlockSpec((T, _D_MODEL), lambda i: (i, 0)),
        out_shape=jax.ShapeDtypeStruct((N, _D_MODEL), jnp.float32),
        compiler_params=pltpu.CompilerParams(
            dimension_semantics=("parallel",)),
    )(ch2d, p)
    return out2d.reshape(B, L, _D_MODEL)


# speedup vs baseline: 1.2536x; 1.2017x over previous
from functools import partial

import numpy as np
import jax
import jax.numpy as jnp
from jax.experimental import pallas as pl
from jax.experimental.pallas import tpu as pltpu

_D_MODEL = 1024
_VALUE_RANGE = 4
_ROW_TILE = 2048


def _pos_code_sum(k, d_model):
    positions = np.arange(k, dtype=np.float64)[:, None]
    i = np.arange(0, d_model, 2, dtype=np.float64)
    omega = 1.0 / (10000.0 ** (i / d_model))
    angles = positions * omega[None, :]
    codes = np.zeros((k, d_model), np.float64)
    codes[:, 0::2] = np.sin(angles)
    codes[:, 1::2] = np.cos(angles)
    return codes.sum(axis=0).astype(np.float32)


def _body(ch_ref, ve_ref, ps_ref, b_ref, mix_ref, w_ref, o_ref, p_ref,
          *, tile, k, slab=8):
    d = ve_ref.shape[1]

    @pl.when(pl.program_id(0) == 0)
    def _prep():
        a = jnp.concatenate(
            [ve_ref[...], ps_ref[...], jnp.zeros((3, d), jnp.float32)], axis=0)
        raw = jax.lax.dot_general(a, w_ref[...], (((1,), (1,)), ((), ())),
                                  preferred_element_type=jnp.float32)
        mix = mix_ref[0]
        base = mix * (raw[4:5] + b_ref[...] + float(k) * raw[0:1])
        deltas = mix * (raw[1:4] - raw[0:1])
        p_ref[...] = jnp.concatenate(
            [base, deltas, jnp.zeros((4, d), jnp.float32)], axis=0)

    base = jnp.broadcast_to(p_ref[0:1, :], (slab, d))
    d1 = jnp.broadcast_to(p_ref[1:2, :], (slab, d))
    d2 = jnp.broadcast_to(p_ref[2:3, :], (slab, d))
    d3 = jnp.broadcast_to(p_ref[3:4, :], (slab, d))

    for i in range(tile // slab):
        r = i * slab
        ch = jnp.clip(ch_ref[pl.ds(r, slab), :], 0, _VALUE_RANGE - 1)
        c1 = jnp.sum((ch == 1).astype(jnp.float32), axis=1, keepdims=True)
        c2 = jnp.sum((ch == 2).astype(jnp.float32), axis=1, keepdims=True)
        c3 = jnp.sum((ch == 3).astype(jnp.float32), axis=1, keepdims=True)
        o_ref[pl.ds(r, slab), :] = base + c1 * d1 + c2 * d2 + c3 * d3


def kernel(channels, value_emb, read_W, read_b, mix):
    B, L, K = channels.shape
    N = B * L
    ch2d = channels.reshape(N, K)
    pos_sum = jnp.asarray(_pos_code_sum(K, _D_MODEL))[None, :]
    b2d = read_b[None, :]
    mix1 = jnp.asarray(mix, jnp.float32).reshape(1)

    T = _ROW_TILE
    whole = lambda i: (0, 0)
    out2d = pl.pallas_call(
        partial(_body, tile=T, k=K),
        grid=(N // T,),
        in_specs=[pl.BlockSpec((T, K), lambda i: (i, 0)),
                  pl.BlockSpec((_VALUE_RANGE, _D_MODEL), whole),
                  pl.BlockSpec((1, _D_MODEL), whole),
                  pl.BlockSpec((1, _D_MODEL), whole),
                  pl.BlockSpec(memory_space=pltpu.SMEM),
                  pl.BlockSpec((_D_MODEL, _D_MODEL), whole)],
        out_specs=pl.BlockSpec((T, _D_MODEL), lambda i: (i, 0)),
        out_shape=jax.ShapeDtypeStruct((N, _D_MODEL), jnp.float32),
        scratch_shapes=[pltpu.VMEM((8, _D_MODEL), jnp.float32)],
        compiler_params=pltpu.CompilerParams(
            dimension_semantics=("arbitrary",)),
    )(ch2d, value_emb, pos_sum, b2d, mix1, read_W)
    return out2d.reshape(B, L, _D_MODEL)
